# initial kernel scaffold (unmeasured)
import jax
import jax.numpy as jnp
from jax import lax
from jax.experimental import pallas as pl
from jax.experimental.pallas import tpu as pltpu

NC = 8


def kernel(A, B):
    m, k = A.shape
    k2, n = B.shape
    assert k == k2
    assert m % NC == 0
    mc = m // NC

    def body(a_ref, b_ref, out_ref, recv_ref, send_sems, recv_sems):
        my_x = lax.axis_index("x")
        my_y = lax.axis_index("y")
        nbr = (my_x, 1 - my_y)

        barrier_sem = pltpu.get_barrier_semaphore()
        pl.semaphore_signal(
            barrier_sem, inc=1, device_id=nbr,
            device_id_type=pl.DeviceIdType.MESH,
        )
        pl.semaphore_wait(barrier_sem, 1)

        for c in range(NC):
            sl = pl.ds(c * mc, mc)
            slot = c % 2
            out_ref[sl, :] = jnp.dot(
                a_ref[sl, :], b_ref[:, :], preferred_element_type=jnp.float32
            )
            rdma = pltpu.make_async_remote_copy(
                src_ref=out_ref.at[sl, :],
                dst_ref=recv_ref.at[slot],
                send_sem=send_sems.at[slot],
                recv_sem=recv_sems.at[slot],
                device_id=nbr,
                device_id_type=pl.DeviceIdType.MESH,
            )
            rdma.start()
            rdma.wait()
            out_ref[sl, :] = out_ref[sl, :] + recv_ref[slot, :, :]

    return pl.pallas_call(
        body,
        out_shape=jax.ShapeDtypeStruct((m, n), jnp.float32),
        in_specs=[
            pl.BlockSpec(memory_space=pltpu.VMEM),
            pl.BlockSpec(memory_space=pltpu.VMEM),
        ],
        out_specs=pl.BlockSpec(memory_space=pltpu.VMEM),
        scratch_shapes=[
            pltpu.VMEM((2, mc, n), jnp.float32),
            pltpu.SemaphoreType.DMA((2,)),
            pltpu.SemaphoreType.DMA((2,)),
        ],
        compiler_params=pltpu.CompilerParams(collective_id=0),
    )(A, B)


# baseline (device time: 513114 ns/iter reference)
import jax
import jax.numpy as jnp
from jax import lax
from jax.experimental import pallas as pl
from jax.experimental.pallas import tpu as pltpu

NC = 8


def kernel(A, B):
    m, k = A.shape
    k2, n = B.shape
    assert k == k2
    assert m % NC == 0
    mc = m // NC

    def body(a_ref, b_ref, out_ref, acc_buf, recv_buf,
             store_sems, send_sems, recv_sems):
        my_x = lax.axis_index("x")
        my_y = lax.axis_index("y")
        nbr = (my_x, 1 - my_y)

        barrier_sem = pltpu.get_barrier_semaphore()
        pl.semaphore_signal(
            barrier_sem, inc=1, device_id=nbr,
            device_id_type=pl.DeviceIdType.MESH,
        )
        pl.semaphore_wait(barrier_sem, 1)

        for c in range(NC):
            sl = pl.ds(c * mc, mc)
            slot = c % 2
            acc_buf[slot] = jnp.dot(
                a_ref[sl, :], b_ref[:, :], preferred_element_type=jnp.float32
            )
            rdma = pltpu.make_async_remote_copy(
                src_ref=acc_buf.at[slot],
                dst_ref=recv_buf.at[slot],
                send_sem=send_sems.at[slot],
                recv_sem=recv_sems.at[slot],
                device_id=nbr,
                device_id_type=pl.DeviceIdType.MESH,
            )
            rdma.start()
            rdma.wait()
            acc_buf[slot] = acc_buf[slot] + recv_buf[slot]
            store = pltpu.make_async_copy(
                acc_buf.at[slot], out_ref.at[sl, :], store_sems.at[slot]
            )
            store.start()
            store.wait()

    return pl.pallas_call(
        body,
        out_shape=jax.ShapeDtypeStruct((m, n), jnp.float32),
        in_specs=[
            pl.BlockSpec(memory_space=pltpu.VMEM),
            pl.BlockSpec(memory_space=pltpu.VMEM),
        ],
        out_specs=pl.BlockSpec(memory_space=pltpu.MemorySpace.HBM),
        scratch_shapes=[
            pltpu.VMEM((2, mc, n), jnp.float32),
            pltpu.VMEM((2, mc, n), jnp.float32),
            pltpu.SemaphoreType.DMA((2,)),
            pltpu.SemaphoreType.DMA((2,)),
            pltpu.SemaphoreType.DMA((2,)),
        ],
        compiler_params=pltpu.CompilerParams(
            collective_id=0, vmem_limit_bytes=60 * 1024 * 1024
        ),
    )(A, B)


# device time: 266212 ns/iter; 1.9275x vs baseline; 1.9275x over previous
import jax
import jax.numpy as jnp
from jax import lax
from jax.experimental import pallas as pl
from jax.experimental.pallas import tpu as pltpu

NC = 8


def kernel(A, B):
    m, k = A.shape
    k2, n = B.shape
    assert k == k2
    assert m % NC == 0
    mc = m // NC

    def body(a_ref, b_ref, out_ref, acc_buf, send_buf, recv_buf,
             store_sems, send_sems, recv_sems):
        my_x = lax.axis_index("x")
        my_y = lax.axis_index("y")
        nbr = (my_x, 1 - my_y)

        barrier_sem = pltpu.get_barrier_semaphore()
        pl.semaphore_signal(
            barrier_sem, inc=1, device_id=nbr,
            device_id_type=pl.DeviceIdType.MESH,
        )
        pl.semaphore_wait(barrier_sem, 1)

        def dot_chunk(c):
            sl = pl.ds(c * mc, mc)
            acc_buf[c % 2] = jnp.dot(
                a_ref[sl, :], b_ref[:, :], preferred_element_type=jnp.float32
            )

        def cast_chunk(c):
            send_buf[c % 2] = acc_buf[c % 2].astype(jnp.bfloat16)

        rdmas = []
        stores = []

        dot_chunk(0)
        cast_chunk(0)

        for c in range(NC):
            slot = c % 2
            rdma = pltpu.make_async_remote_copy(
                src_ref=send_buf.at[slot],
                dst_ref=recv_buf.at[slot],
                send_sem=send_sems.at[slot],
                recv_sem=recv_sems.at[slot],
                device_id=nbr,
                device_id_type=pl.DeviceIdType.MESH,
            )
            rdmas.append(rdma)
            rdma.start()
            if c + 1 < NC:
                if c >= 1:
                    stores[c - 1].wait()
                dot_chunk(c + 1)
                if c >= 1:
                    rdmas[c - 1].wait_send()
                cast_chunk(c + 1)
            rdma.wait_recv()
            acc_buf[slot] = acc_buf[slot] + recv_buf[slot].astype(jnp.float32)
            store = pltpu.make_async_copy(
                acc_buf.at[slot],
                out_ref.at[pl.ds(c * mc, mc), :],
                store_sems.at[slot],
            )
            stores.append(store)
            store.start()

        rdmas[NC - 2].wait_send()
        rdmas[NC - 1].wait_send()
        stores[NC - 2].wait()
        stores[NC - 1].wait()

    return pl.pallas_call(
        body,
        out_shape=jax.ShapeDtypeStruct((m, n), jnp.float32),
        in_specs=[
            pl.BlockSpec(memory_space=pltpu.VMEM),
            pl.BlockSpec(memory_space=pltpu.VMEM),
        ],
        out_specs=pl.BlockSpec(memory_space=pltpu.MemorySpace.HBM),
        scratch_shapes=[
            pltpu.VMEM((2, mc, n), jnp.float32),
            pltpu.VMEM((2, mc, n), jnp.bfloat16),
            pltpu.VMEM((2, mc, n), jnp.bfloat16),
            pltpu.SemaphoreType.DMA((2,)),
            pltpu.SemaphoreType.DMA((2,)),
            pltpu.SemaphoreType.DMA((2,)),
        ],
        compiler_params=pltpu.CompilerParams(
            collective_id=0, vmem_limit_bytes=60 * 1024 * 1024
        ),
    )(A, B)


# device time: 250353 ns/iter; 2.0496x vs baseline; 1.0633x over previous
import jax
import jax.numpy as jnp
from jax import lax
from jax.experimental import pallas as pl
from jax.experimental.pallas import tpu as pltpu

NC = 8


def kernel(A, B):
    m, k = A.shape
    k2, n = B.shape
    assert k == k2
    assert m % NC == 0
    mc = m // NC

    def body(a_ref, b_ref, out_ref, acc_buf, send_buf, recv_buf,
             store_sems, send_sems, recv_sems):
        my_x = lax.axis_index("x")
        my_y = lax.axis_index("y")
        nbr = (my_x, 1 - my_y)

        barrier_sem = pltpu.get_barrier_semaphore()
        pl.semaphore_signal(
            barrier_sem, inc=1, device_id=nbr,
            device_id_type=pl.DeviceIdType.MESH,
        )
        pl.semaphore_wait(barrier_sem, 1)

        def dot_chunk(c):
            sl = pl.ds(c * mc, mc)
            acc_buf[c % 2] = jnp.dot(
                a_ref[sl, :], b_ref[:, :], preferred_element_type=jnp.float32
            )

        def cast_chunk(c):
            send_buf[c % 2] = acc_buf[c % 2].astype(jnp.bfloat16)

        def make_rdma(c):
            return pltpu.make_async_remote_copy(
                src_ref=send_buf.at[c % 2],
                dst_ref=recv_buf.at[c % 4],
                send_sem=send_sems.at[c % 2],
                recv_sem=recv_sems.at[c % 4],
                device_id=nbr,
                device_id_type=pl.DeviceIdType.MESH,
            )

        rdmas = []
        stores = []

        dot_chunk(0)
        cast_chunk(0)
        rdmas.append(make_rdma(0))
        rdmas[0].start()

        for c in range(NC):
            slot = c % 2
            if c + 1 < NC:
                if c >= 1:
                    stores[c - 1].wait()
                dot_chunk(c + 1)
                if c >= 1:
                    rdmas[c - 1].wait_send()
                cast_chunk(c + 1)
                rdmas.append(make_rdma(c + 1))
                rdmas[c + 1].start()
            rdmas[c].wait_recv()
            acc_buf[slot] = acc_buf[slot] + recv_buf[c % 4].astype(jnp.float32)
            store = pltpu.make_async_copy(
                acc_buf.at[slot],
                out_ref.at[pl.ds(c * mc, mc), :],
                store_sems.at[slot],
            )
            stores.append(store)
            store.start()

        rdmas[NC - 2].wait_send()
        rdmas[NC - 1].wait_send()
        stores[NC - 2].wait()
        stores[NC - 1].wait()

    return pl.pallas_call(
        body,
        out_shape=jax.ShapeDtypeStruct((m, n), jnp.float32),
        in_specs=[
            pl.BlockSpec(memory_space=pltpu.VMEM),
            pl.BlockSpec(memory_space=pltpu.VMEM),
        ],
        out_specs=pl.BlockSpec(memory_space=pltpu.MemorySpace.HBM),
        scratch_shapes=[
            pltpu.VMEM((2, mc, n), jnp.float32),
            pltpu.VMEM((2, mc, n), jnp.bfloat16),
            pltpu.VMEM((4, mc, n), jnp.bfloat16),
            pltpu.SemaphoreType.DMA((2,)),
            pltpu.SemaphoreType.DMA((2,)),
            pltpu.SemaphoreType.DMA((4,)),
        ],
        compiler_params=pltpu.CompilerParams(
            collective_id=0, vmem_limit_bytes=60 * 1024 * 1024
        ),
    )(A, B)
